# Initial kernel scaffold; baseline (speedup 1.0000x reference)
import jax
import jax.numpy as jnp
from jax.experimental import pallas as pl


def kernel(bins, weights, n_samples):
    raise NotImplementedError("write your pallas kernel here")


# speedup vs baseline: 26.9013x; 26.9013x over previous
import functools

import jax
import jax.numpy as jnp
from jax import lax
from jax.experimental import pallas as pl
from jax.experimental.pallas import tpu as pltpu
from jax.experimental.pallas import tpu_sc as plsc

N_R = 65536
N_B = 128
N_S = 128
L = 16
N_WORKERS = 32
NBP = N_B + 1


def _sc_body(bins_hbm, w_hbm, out_hbm, bins_v, w_v, a_v, b_v, out_v):
    cid = lax.axis_index("c")
    sid = lax.axis_index("s")
    wid = sid * 2 + cid
    rays_per_w = N_R // N_WORKERS
    groups = rays_per_w // L

    lane = lax.broadcasted_iota(jnp.int32, (L,), 0)
    laneW = lane * N_B
    laneB = lane * NBP
    laneS = lane * N_S
    zero16 = jnp.zeros((L,), jnp.float32)

    def _init(j, c):
        a_v[pl.ds(j * L, L)] = zero16
        b_v[pl.ds(j * L, L)] = zero16
        return c
    lax.fori_loop(0, N_S, _init, 0)

    def group_body(g, carry):
        r0 = (wid * rays_per_w + g * L).astype(jnp.int32)
        pltpu.sync_copy(bins_hbm.at[pl.ds(r0 * NBP, L * NBP)], bins_v)
        pltpu.sync_copy(w_hbm.at[pl.ds(r0 * N_B, L * N_B)], w_v)

        def p1(i, acc):
            ivec = jnp.full((L,), i, jnp.int32)
            col = plsc.load_gather(w_v, [laneW + ivec])
            return acc + (col + 1e-5)
        total = lax.fori_loop(0, N_B, p1, zero16)
        inv_t = 1.0 / total

        bins0 = plsc.load_gather(bins_v, [laneB])

        def scatter(jpos, dA, dB):
            m = jpos < N_S
            jsafe = jnp.where(m, jpos, 0)
            idx = jsafe * L + lane
            plsc.addupdate_scatter(a_v, [idx], dA, mask=m)
            plsc.addupdate_scatter(b_v, [idx], dB, mask=m)

        def jpos_of(cdf0):
            y = cdf0 * 128.0 - 0.5
            ti = y.astype(jnp.int32)
            tf = ti.astype(jnp.float32)
            j = ti + jnp.where(y > tf, 1, 0)
            return jnp.maximum(j, 0)

        def p2(i, c):
            acc, cdf0, bi, A_prev, B_prev = c
            ivec = jnp.full((L,), i, jnp.int32)
            wcol = plsc.load_gather(w_v, [laneW + ivec])
            acc = acc + (wcol + 1e-5)
            cdf1 = acc * inv_t
            bi1 = plsc.load_gather(bins_v, [laneB + ivec + 1])
            denom = cdf1 - cdf0
            denom = jnp.where(denom < 1e-5, 1.0, denom)
            B = (bi1 - bi) / denom
            A = bi - cdf0 * B
            scatter(jpos_of(cdf0), A - A_prev, B - B_prev)
            return (acc, cdf1, bi1, A, B)

        c = (zero16, zero16, bins0, zero16, zero16)
        _, cdfN, binsN, A_prev, B_prev = lax.fori_loop(0, N_B, p2, c)
        scatter(jpos_of(cdfN), binsN - A_prev, -B_prev)

        def p3(j, c3):
            aa, ab = c3
            row = pl.ds(j * L, L)
            aa = aa + a_v[row]
            ab = ab + b_v[row]
            a_v[row] = zero16
            b_v[row] = zero16
            jvec = jnp.full((L,), j, jnp.int32)
            u = (jvec.astype(jnp.float32) + 0.5) * (1.0 / 128.0)
            val = aa + u * ab
            plsc.store_scatter(out_v, [laneS + jvec], val)
            return (aa, ab)
        lax.fori_loop(0, N_S, p3, (zero16, zero16))

        pltpu.sync_copy(out_v, out_hbm.at[pl.ds(r0 * N_S, L * N_S)])
        return carry

    lax.fori_loop(0, groups, group_body, 0)


@jax.jit
def _run(bins, weights):
    mesh = plsc.VectorSubcoreMesh(core_axis_name="c", subcore_axis_name="s")
    kfn = pl.kernel(
        _sc_body,
        out_type=jax.ShapeDtypeStruct((N_R * N_S,), jnp.float32),
        mesh=mesh,
        compiler_params=pltpu.CompilerParams(needs_layout_passes=False),
        scratch_types=[
            pltpu.VMEM((L * NBP,), jnp.float32),
            pltpu.VMEM((L * N_B,), jnp.float32),
            pltpu.VMEM((N_S * L,), jnp.float32),
            pltpu.VMEM((N_S * L,), jnp.float32),
            pltpu.VMEM((L * N_S,), jnp.float32),
        ],
    )
    out = kfn(bins.reshape(-1), weights.reshape(-1))
    return out.reshape(N_R, N_S)


def kernel(bins, weights, n_samples):
    return _run(bins, weights)


# speedup vs baseline: 33.8002x; 1.2565x over previous
import functools

import jax
import jax.numpy as jnp
from jax import lax
from jax.experimental import pallas as pl
from jax.experimental.pallas import tpu as pltpu
from jax.experimental.pallas import tpu_sc as plsc

N_R = 65536
N_B = 128
N_S = 128
L = 16
N_WORKERS = 32
NBP = N_B + 1


def _sc_body(bins_hbm, w_hbm, out_hbm, bins_v, w_v, a_v, b_v, out_v):
    cid = lax.axis_index("c")
    sid = lax.axis_index("s")
    wid = sid * 2 + cid
    rays_per_w = N_R // N_WORKERS
    groups = rays_per_w // L

    lane = lax.broadcasted_iota(jnp.int32, (L,), 0)
    laneP = lane * NBP
    zero16 = jnp.zeros((L,), jnp.float32)
    one16i = jnp.ones((L,), jnp.int32)

    def _init(j, c):
        a_v[pl.ds(j * L, L)] = zero16
        b_v[pl.ds(j * L, L)] = zero16
        return c
    lax.fori_loop(0, N_S, _init, 0)

    def group_body(g, carry):
        r0 = (wid * rays_per_w + g * L).astype(jnp.int32)
        pltpu.sync_copy(bins_hbm.at[pl.ds(r0 * NBP, L * NBP)], bins_v)
        pltpu.sync_copy(w_hbm.at[pl.ds(r0 * NBP, L * NBP)], w_v)

        def p1(i, c):
            acc, idxv = c
            col = plsc.load_gather(w_v, [idxv])
            return (acc + (col + 1e-5), idxv + one16i)
        total, _ = lax.fori_loop(0, N_B, p1, (zero16, laneP), unroll=4)
        inv_t = 1.0 / total

        bins0 = plsc.load_gather(bins_v, [laneP])

        def scatter(jpos, dA, dB):
            m = jpos < N_S
            jsafe = jnp.where(m, jpos, 0)
            idx = jsafe * L + lane
            plsc.addupdate_scatter(a_v, [idx], dA, mask=m)
            plsc.addupdate_scatter(b_v, [idx], dB, mask=m)

        def jpos_of(cdf0):
            y = cdf0 * 128.0 - 0.5
            ti = y.astype(jnp.int32)
            tf = ti.astype(jnp.float32)
            j = ti + jnp.where(y > tf, 1, 0)
            return jnp.maximum(j, 0)

        def p2(i, c):
            acc, cdf0, bi, A_prev, B_prev, idxv = c
            wcol = plsc.load_gather(w_v, [idxv])
            acc = acc + (wcol + 1e-5)
            cdf1 = acc * inv_t
            bi1 = plsc.load_gather(bins_v, [idxv + one16i])
            denom = cdf1 - cdf0
            denom = jnp.where(denom < 1e-5, 1.0, denom)
            B = (bi1 - bi) / denom
            A = bi - cdf0 * B
            scatter(jpos_of(cdf0), A - A_prev, B - B_prev)
            return (acc, cdf1, bi1, A, B, idxv + one16i)

        c = (zero16, zero16, bins0, zero16, zero16, laneP)
        _, cdfN, binsN, A_prev, B_prev, _ = lax.fori_loop(0, N_B, p2, c, unroll=4)
        scatter(jpos_of(cdfN), binsN - A_prev, -B_prev)

        def p3(j, c3):
            aa, ab, idxv, u = c3
            row = pl.ds(j * L, L)
            aa = aa + a_v[row]
            ab = ab + b_v[row]
            a_v[row] = zero16
            b_v[row] = zero16
            val = aa + u * ab
            plsc.store_scatter(out_v, [idxv], val)
            return (aa, ab, idxv + one16i, u + (1.0 / 128.0))
        u0 = jnp.full((L,), 0.5 / 128.0, jnp.float32)
        lax.fori_loop(0, N_S, p3, (zero16, zero16, laneP, u0), unroll=4)

        pltpu.sync_copy(out_v, out_hbm.at[pl.ds(r0 * NBP, L * NBP)])
        return carry

    lax.fori_loop(0, groups, group_body, 0)


@jax.jit
def _run(bins, weights):
    mesh = plsc.VectorSubcoreMesh(core_axis_name="c", subcore_axis_name="s")
    kfn = pl.kernel(
        _sc_body,
        out_type=jax.ShapeDtypeStruct((N_R * NBP,), jnp.float32),
        mesh=mesh,
        compiler_params=pltpu.CompilerParams(needs_layout_passes=False),
        scratch_types=[
            pltpu.VMEM((L * NBP,), jnp.float32),
            pltpu.VMEM((L * NBP,), jnp.float32),
            pltpu.VMEM((N_S * L,), jnp.float32),
            pltpu.VMEM((N_S * L,), jnp.float32),
            pltpu.VMEM((L * NBP,), jnp.float32),
        ],
    )
    w_pad = jnp.pad(weights, ((0, 0), (0, 1)))
    out = kfn(bins.reshape(-1), w_pad.reshape(-1))
    return out.reshape(N_R, NBP)[:, :N_S]


def kernel(bins, weights, n_samples):
    return _run(bins, weights)


# speedup vs baseline: 36.9412x; 1.0929x over previous
import functools

import jax
import jax.numpy as jnp
from jax import lax
from jax.experimental import pallas as pl
from jax.experimental.pallas import tpu as pltpu
from jax.experimental.pallas import tpu_sc as plsc

N_R = 65536
N_B = 128
N_S = 128
L = 16
N_WORKERS = 32
NBP = N_B + 1
C = 2
CL = C * L


def _sc_body(bins_hbm, w_hbm, out_hbm, bins_v, w_v, a_v, b_v, out_v):
    cid = lax.axis_index("c")
    sid = lax.axis_index("s")
    wid = sid * 2 + cid
    rays_per_w = N_R // N_WORKERS
    groups = rays_per_w // CL

    lane = lax.broadcasted_iota(jnp.int32, (L,), 0)
    laneP = [lane * NBP + k * (L * NBP) for k in range(C)]
    zero16 = jnp.zeros((L,), jnp.float32)
    one16i = jnp.ones((L,), jnp.int32)

    def _init(j, c):
        a_v[pl.ds(j * L, L)] = zero16
        b_v[pl.ds(j * L, L)] = zero16
        return c
    lax.fori_loop(0, N_S * C, _init, 0)

    def group_body(g, carry):
        r0 = (wid * rays_per_w + g * CL).astype(jnp.int32)
        pltpu.sync_copy(bins_hbm.at[pl.ds(r0 * NBP, CL * NBP)], bins_v)
        pltpu.sync_copy(w_hbm.at[pl.ds(r0 * NBP, CL * NBP)], w_v)

        def p1(i, c):
            out = []
            for k in range(C):
                acc, idxv = c[k]
                col = plsc.load_gather(w_v, [idxv])
                out.append((acc + (col + 1e-5), idxv + one16i))
            return tuple(out)
        tot = lax.fori_loop(0, N_B, p1,
                            tuple((zero16, laneP[k]) for k in range(C)),
                            unroll=4)
        inv_t = [1.0 / tot[k][0] for k in range(C)]

        def scatter(k, jpos, dA, dB):
            m = jpos < N_S
            jsafe = jnp.where(m, jpos, 0)
            idx = jsafe * CL + (k * L) + lane
            plsc.addupdate_scatter(a_v, [idx], dA, mask=m)
            plsc.addupdate_scatter(b_v, [idx], dB, mask=m)

        def jpos_of(cdf0):
            y = cdf0 * 128.0 - 0.5
            ti = y.astype(jnp.int32)
            tf = ti.astype(jnp.float32)
            j = ti + jnp.where(y > tf, 1, 0)
            return jnp.maximum(j, 0)

        def p2(i, c):
            out = []
            for k in range(C):
                acc, cdf0, bi, A_prev, B_prev, idxv = c[k]
                wcol = plsc.load_gather(w_v, [idxv])
                acc = acc + (wcol + 1e-5)
                cdf1 = acc * inv_t[k]
                bi1 = plsc.load_gather(bins_v, [idxv + one16i])
                denom = cdf1 - cdf0
                denom = jnp.where(denom < 1e-5, 1.0, denom)
                B = (bi1 - bi) / denom
                A = bi - cdf0 * B
                scatter(k, jpos_of(cdf0), A - A_prev, B - B_prev)
                out.append((acc, cdf1, bi1, A, B, idxv + one16i))
            return tuple(out)

        bins0 = [plsc.load_gather(bins_v, [laneP[k]]) for k in range(C)]
        cfin = lax.fori_loop(
            0, N_B, p2,
            tuple((zero16, zero16, bins0[k], zero16, zero16, laneP[k])
                  for k in range(C)),
            unroll=4)
        for k in range(C):
            _, cdfN, binsN, A_prev, B_prev, _ = cfin[k]
            scatter(k, jpos_of(cdfN), binsN - A_prev, -B_prev)

        def p3(j, c3):
            u = c3[-1]
            out = []
            for k in range(C):
                aa, ab, idxv = c3[k]
                row = pl.ds(j * CL + k * L, L)
                aa = aa + a_v[row]
                ab = ab + b_v[row]
                a_v[row] = zero16
                b_v[row] = zero16
                val = aa + u * ab
                plsc.store_scatter(out_v, [idxv], val)
                out.append((aa, ab, idxv + one16i))
            out.append(u + (1.0 / 128.0))
            return tuple(out)
        u0 = jnp.full((L,), 0.5 / 128.0, jnp.float32)
        lax.fori_loop(0, N_S, p3,
                      tuple((zero16, zero16, laneP[k]) for k in range(C))
                      + (u0,),
                      unroll=4)

        pltpu.sync_copy(out_v, out_hbm.at[pl.ds(r0 * NBP, CL * NBP)])
        return carry

    lax.fori_loop(0, groups, group_body, 0)


@jax.jit
def _run(bins, weights):
    mesh = plsc.VectorSubcoreMesh(core_axis_name="c", subcore_axis_name="s")
    kfn = pl.kernel(
        _sc_body,
        out_type=jax.ShapeDtypeStruct((N_R * NBP,), jnp.float32),
        mesh=mesh,
        compiler_params=pltpu.CompilerParams(needs_layout_passes=False),
        scratch_types=[
            pltpu.VMEM((CL * NBP,), jnp.float32),
            pltpu.VMEM((CL * NBP,), jnp.float32),
            pltpu.VMEM((N_S * CL,), jnp.float32),
            pltpu.VMEM((N_S * CL,), jnp.float32),
            pltpu.VMEM((CL * NBP,), jnp.float32),
        ],
    )
    w_pad = jnp.pad(weights, ((0, 0), (0, 1)))
    out = kfn(bins.reshape(-1), w_pad.reshape(-1))
    return out.reshape(N_R, NBP)[:, :N_S]


def kernel(bins, weights, n_samples):
    return _run(bins, weights)
